# Initial kernel scaffold; baseline (speedup 1.0000x reference)
import jax
import jax.numpy as jnp
from jax.experimental import pallas as pl


def kernel(token_ids, weight):
    raise NotImplementedError("write your pallas kernel here")


# speedup vs baseline: 1.1015x; 1.1015x over previous
import functools

import jax
import jax.numpy as jnp
from jax import lax
from jax.experimental import pallas as pl
from jax.experimental.pallas import tpu as pltpu
from jax.experimental.pallas import tpu_sc as plsc

_EMBED_DIM = 32
_IDXW = 128


@functools.cache
def _make_gather(total_rows: int, dim: int, vocab: int):
    info = plsc.get_sparse_core_info()
    nw = info.num_cores * info.num_subcores
    rows_per_w = total_rows // nw
    n_idx_rows = rows_per_w // _IDXW
    chunk = 8
    n_chunks = n_idx_rows // chunk

    mesh = plsc.VectorSubcoreMesh(core_axis_name="c", subcore_axis_name="s")

    @functools.partial(
        pl.kernel,
        mesh=mesh,
        out_type=jax.ShapeDtypeStruct((total_rows, dim), jnp.float32),
        scratch_types=[
            pltpu.VMEM((chunk, _IDXW), jnp.int32),
            pltpu.VMEM((chunk * _IDXW, dim), jnp.float32),
            pltpu.SemaphoreType.DMA,
        ],
        compiler_params=pltpu.CompilerParams(use_tc_tiling_on_sc=False),
    )
    def gather_kernel(idx_hbm, table_hbm, out_hbm, idx_v, rows_v, sem):
        wid = lax.axis_index("s") * info.num_cores + lax.axis_index("c")
        row_base = wid * n_idx_rows

        def body(j, carry):
            idx_row = row_base + j * chunk
            pltpu.sync_copy(idx_hbm.at[pl.ds(idx_row, chunk)], idx_v)
            copies = [
                pltpu.async_copy(
                    table_hbm.at[idx_v.at[r]],
                    rows_v.at[pl.ds(r * _IDXW, _IDXW)],
                    sem,
                )
                for r in range(chunk)
            ]
            for c in copies:
                c.wait()
            pltpu.sync_copy(
                rows_v, out_hbm.at[pl.ds(idx_row * _IDXW, chunk * _IDXW)]
            )
            return carry

        lax.fori_loop(0, n_chunks, body, 0)

    return gather_kernel


def kernel(token_ids, weight):
    batch, seq = token_ids.shape
    total = batch * seq
    idx2d = token_ids.reshape(total // _IDXW, _IDXW).astype(jnp.int32)
    flat = _make_gather(total, weight.shape[1], weight.shape[0])(idx2d, weight)
    return flat.reshape(batch, seq, weight.shape[1])


# speedup vs baseline: 1.1125x; 1.0100x over previous
import functools

import jax
import jax.numpy as jnp
from jax import lax
from jax.experimental import pallas as pl
from jax.experimental.pallas import tpu as pltpu
from jax.experimental.pallas import tpu_sc as plsc

_EMBED_DIM = 32
_IDXW = 128
_CHUNK = 8


@functools.cache
def _make_gather(total_rows: int, dim: int):
    info = plsc.get_sparse_core_info()
    nw = info.num_cores * info.num_subcores
    rows_per_w = total_rows // nw
    n_idx_rows = rows_per_w // _IDXW
    n_chunks = n_idx_rows // _CHUNK
    rows_per_chunk = _CHUNK * _IDXW

    mesh = plsc.VectorSubcoreMesh(core_axis_name="c", subcore_axis_name="s")

    @functools.partial(
        pl.kernel,
        mesh=mesh,
        out_type=jax.ShapeDtypeStruct((total_rows, dim), jnp.float32),
        scratch_types=[
            pltpu.VMEM((n_idx_rows, _IDXW), jnp.int32),
            pltpu.VMEM((2, rows_per_chunk, dim), jnp.float32),
            pltpu.SemaphoreType.DMA,
            pltpu.SemaphoreType.DMA,
            pltpu.SemaphoreType.DMA,
            pltpu.SemaphoreType.DMA,
        ],
        compiler_params=pltpu.CompilerParams(use_tc_tiling_on_sc=False),
    )
    def gather_kernel(idx_hbm, table_hbm, out_hbm, idx_v, rows_v, g0, g1, o0, o1):
        wid = lax.axis_index("s") * info.num_cores + lax.axis_index("c")
        row_base = wid * n_idx_rows
        gsem = (g0, g1)
        osem = (o0, o1)

        pltpu.sync_copy(idx_hbm.at[pl.ds(row_base, n_idx_rows)], idx_v)

        def fire_gathers(chunk, slot, sem):
            for r in range(_CHUNK):
                pltpu.async_copy(
                    table_hbm.at[idx_v.at[chunk * _CHUNK + r]],
                    rows_v.at[slot, pl.ds(r * _IDXW, _IDXW)],
                    sem,
                )

        def drain_gathers(slot, sem):
            pltpu.make_async_copy(
                out_hbm.at[pl.ds(0, rows_per_chunk)], rows_v.at[slot], sem
            ).wait()

        def writeback(chunk, slot, sem):
            out_row = (row_base * _IDXW) + chunk * rows_per_chunk
            pltpu.async_copy(
                rows_v.at[slot], out_hbm.at[pl.ds(out_row, rows_per_chunk)], sem
            ).wait()

        fire_gathers(0, 0, gsem[0])
        fire_gathers(1, 1, gsem[1])

        def body(j0, carry):
            for b in range(2):
                cur = j0 * 2 + b
                drain_gathers(b, gsem[b])
                writeback(cur, b, osem[b])
                fire_gathers(cur + 2, b, gsem[b])
            return carry

        lax.fori_loop(0, (n_chunks - 2) // 2, body, 0)

        for b in range(2):
            cur = n_chunks - 2 + b
            drain_gathers(b, gsem[b])
            writeback(cur, b, osem[b])

    return gather_kernel


def kernel(token_ids, weight):
    batch, seq = token_ids.shape
    total = batch * seq
    idx2d = token_ids.reshape(total // _IDXW, _IDXW).astype(jnp.int32)
    flat = _make_gather(total, weight.shape[1])(idx2d, weight)
    return flat.reshape(batch, seq, weight.shape[1])


# speedup vs baseline: 3.4995x; 3.1456x over previous
import functools

import jax
import jax.numpy as jnp
from jax import lax
from jax.experimental import pallas as pl
from jax.experimental.pallas import tpu as pltpu
from jax.experimental.pallas import tpu_sc as plsc

_LANES = 16
_BT = 128


@functools.cache
def _make_lookup(seq, batch, dim, vocab):
    info = plsc.get_sparse_core_info()
    nw = info.num_cores * info.num_subcores
    tiles_per_w = batch // _BT // nw
    bpw = tiles_per_w * _BT
    n_units = seq * tiles_per_w
    dtiles = dim // 8

    mesh = plsc.VectorSubcoreMesh(core_axis_name="c", subcore_axis_name="s")

    @functools.partial(
        pl.kernel,
        mesh=mesh,
        out_type=jax.ShapeDtypeStruct((seq, dtiles, batch // _BT, 8, _BT),
                                      jnp.float32),
        scratch_types=[
            pltpu.VMEM((seq, bpw), jnp.int32),
            pltpu.VMEM((2, _BT, dim), jnp.float32),
            pltpu.VMEM((2, dtiles, 8, _BT), jnp.float32),
            pltpu.SemaphoreType.DMA,
            pltpu.SemaphoreType.DMA,
            pltpu.SemaphoreType.DMA,
            pltpu.SemaphoreType.DMA,
        ],
        compiler_params=pltpu.CompilerParams(
            use_tc_tiling_on_sc=False, needs_layout_passes=False
        ),
    )
    def lookup_kernel(tt_hbm, table_hbm, out_hbm, idx_v, gbuf, tbuf,
                      g0, g1, o0, o1):
        wid = lax.axis_index("s") * info.num_cores + lax.axis_index("c")
        tile0 = wid * tiles_per_w
        gsem = (g0, g1)
        osem = (o0, o1)

        pltpu.sync_copy(tt_hbm.at[:, pl.ds(tile0 * _BT, bpw)], idx_v)

        row_iota = lax.iota(jnp.int32, _LANES)

        def fire_gather(u, slot, sem):
            s = u // tiles_per_w
            j = lax.rem(u, tiles_per_w)
            pltpu.async_copy(
                table_hbm.at[idx_v.at[s, pl.ds(j * _BT, _BT)]],
                gbuf.at[slot],
                sem,
            )

        def drain_gather(slot, sem):
            pltpu.make_async_copy(
                table_hbm.at[pl.ds(0, _BT)], gbuf.at[slot], sem
            ).wait()

        def drain_write(slot, sem):
            pltpu.make_async_copy(
                tbuf.at[slot], out_hbm.at[0, :, 0], sem
            ).wait()

        def transpose(slot):
            src = gbuf.at[slot]
            dst = tbuf.at[slot]
            for d in range(dim):
                for ci in range(_BT // _LANES):
                    vec = plsc.load_gather(
                        src,
                        [row_iota + (ci * _LANES),
                         jnp.full((_LANES,), d, jnp.int32)],
                    )
                    dst[d // 8, d % 8, pl.ds(ci * _LANES, _LANES)] = vec

        def fire_write(u, slot, sem):
            s = u // tiles_per_w
            j = lax.rem(u, tiles_per_w)
            pltpu.async_copy(
                tbuf.at[slot],
                out_hbm.at[s, :, tile0 + j],
                sem,
            )

        fire_gather(0, 0, gsem[0])
        fire_gather(1, 1, gsem[1])

        def body(u0, carry):
            for b in range(2):
                u = u0 * 2 + b
                drain_gather(b, gsem[b])

                @pl.when(u >= 2)
                def _():
                    drain_write(b, osem[b])

                transpose(b)
                fire_write(u, b, osem[b])
                fire_gather(u + 2, b, gsem[b])
            return carry

        lax.fori_loop(0, (n_units - 2) // 2, body, 0)

        for b in range(2):
            u = n_units - 2 + b
            drain_gather(b, gsem[b])
            drain_write(b, osem[b])
            transpose(b)
            fire_write(u, b, osem[b])
            drain_write(b, osem[b])

    return lookup_kernel


def kernel(token_ids, weight):
    batch, seq = token_ids.shape
    vocab, dim = weight.shape
    tt = token_ids.T
    out5 = _make_lookup(seq, batch, dim, vocab)(tt, weight)
    return out5.transpose(2, 4, 0, 1, 3).reshape(batch, seq, dim)
